# baseline (speedup 1.0000x reference)
import jax
import jax.numpy as jnp
from jax import lax
from jax.experimental import pallas as pl
from jax.experimental.pallas import tpu as pltpu
from jax.experimental.pallas import tpu_sc as plsc

_L = 16


def _pos_body(col_hbm, row_hbm, out_hbm, col_v, row_v, content_v, sem):
    n_ch = content_v.shape[0]
    h, w = content_v.shape[1], content_v.shape[2]
    batches = out_hbm.shape[0]

    pltpu.sync_copy(col_hbm, col_v)
    pltpu.sync_copy(row_hbm, row_v)

    wid = lax.axis_index("s") * 2 + lax.axis_index("c")
    is_x = wid < 16
    c0 = lax.rem(wid, 16) * n_ch

    iota = lax.iota(jnp.int32, _L)
    zeros = jnp.zeros((_L,), jnp.int32)
    cidx = [zeros + (c0 + ci) for ci in range(n_ch)]

    @pl.when(is_x)
    def _():
        for ci in range(n_ch):
            v_lo = plsc.load_gather(col_v, [iota, cidx[ci]])
            v_hi = plsc.load_gather(col_v, [iota + _L, cidx[ci]])

            def fill(y, carry, ci=ci, v_lo=v_lo, v_hi=v_hi):
                content_v[ci, y, pl.ds(0, _L)] = v_lo
                content_v[ci, y, pl.ds(_L, _L)] = v_hi
                return carry

            lax.fori_loop(0, h, fill, 0)

    @pl.when(jnp.logical_not(is_x))
    def _():
        def fill(y, carry):
            yidx = zeros + y
            for ci in range(n_ch):
                v = plsc.load_gather(row_v, [yidx, cidx[ci]])
                content_v[ci, y, pl.ds(0, _L)] = v
                content_v[ci, y, pl.ds(_L, _L)] = v
            return carry

        lax.fori_loop(0, h, fill, 0)

    copies = []
    for b in range(batches):
        dst = out_hbm.at[b, pl.ds(wid * n_ch, n_ch)]
        copies.append(pltpu.async_copy(content_v, dst, sem))
    for cp in copies:
        cp.wait()


@jax.jit
def kernel(pixel_values, row_embeddings, column_embeddings):
    b = pixel_values.shape[0]
    h, w = pixel_values.shape[-2], pixel_values.shape[-1]
    d = column_embeddings.shape[-1]

    run = pl.kernel(
        _pos_body,
        out_type=jax.ShapeDtypeStruct((b, 2 * d, h, w), jnp.float32),
        mesh=plsc.VectorSubcoreMesh(core_axis_name="c", subcore_axis_name="s"),
        compiler_params=pltpu.CompilerParams(
            use_tc_tiling_on_sc=False, needs_layout_passes=False
        ),
        scratch_types=[
            pltpu.VMEM(column_embeddings.shape, jnp.float32),
            pltpu.VMEM(row_embeddings.shape, jnp.float32),
            pltpu.VMEM((16, 32, 32), jnp.float32),
            pltpu.SemaphoreType.DMA,
        ],
    )
    return run(column_embeddings, row_embeddings)


# speedup vs baseline: 3.7165x; 3.7165x over previous
import jax
import jax.numpy as jnp
from jax import lax
from jax.experimental import pallas as pl
from jax.experimental.pallas import tpu as pltpu
from jax.experimental.pallas import tpu_sc as plsc

_L = 16


def _pos_body(col_hbm, row_hbm, out_hbm, col_stage, rowrep_v, row_v, sem, sem2):
    batches, h = out_hbm.shape[0], out_hbm.shape[1]
    n_xg = out_hbm.shape[2]
    n_cg = out_hbm.shape[3]
    n_cgh = n_cg // 2
    y_per = h // 4

    wid = lax.axis_index("s") * 2 + lax.axis_index("c")
    b = wid // 4
    yg = lax.rem(wid, 4)

    col_copies = []
    for xg in range(n_xg):
        for cgl in range(n_cgh):
            src = col_hbm.at[pl.ds(xg * 8, 8), pl.ds(cgl * 128, 128)]
            col_copies.append(pltpu.async_copy(src, col_stage.at[xg, cgl], sem2))

    pltpu.sync_copy(row_hbm.at[pl.ds(yg * y_per, y_per)], row_v)

    def fill(j, carry):
        for cgh in range(n_cgh):
            for ch in range(128 // _L):
                v = row_v[j, pl.ds(cgh * 128 + ch * _L, _L)]
                for xi in range(8):
                    rowrep_v[j, cgh, xi, pl.ds(ch * _L, _L)] = v
        return carry

    lax.fori_loop(0, y_per, fill, 0)

    for cp in col_copies:
        cp.wait()

    out_copies = []
    for j in range(y_per):
        y = yg * y_per + j
        for xg in range(n_xg):
            dst_c = out_hbm.at[b, y, xg, pl.ds(0, n_cgh)]
            out_copies.append(pltpu.async_copy(col_stage.at[xg], dst_c, sem))
            dst_r = out_hbm.at[b, y, xg, pl.ds(n_cgh, n_cgh)]
            out_copies.append(pltpu.async_copy(rowrep_v.at[j], dst_r, sem))
    for cp in out_copies:
        cp.wait()


@jax.jit
def kernel(pixel_values, row_embeddings, column_embeddings):
    b = pixel_values.shape[0]
    h, w = pixel_values.shape[-2], pixel_values.shape[-1]
    d = column_embeddings.shape[-1]
    n_xg, n_cg = w // 8, (2 * d) // 128

    run = pl.kernel(
        _pos_body,
        out_type=jax.ShapeDtypeStruct((b, h, n_xg, n_cg, 8, 128), jnp.float32),
        mesh=plsc.VectorSubcoreMesh(core_axis_name="c", subcore_axis_name="s"),
        compiler_params=pltpu.CompilerParams(
            use_tc_tiling_on_sc=False, needs_layout_passes=False
        ),
        scratch_types=[
            pltpu.VMEM((n_xg, n_cg // 2, 8, 128), jnp.float32),
            pltpu.VMEM((h // 4, n_cg // 2, 8, 128), jnp.float32),
            pltpu.VMEM((h // 4, d), jnp.float32),
            pltpu.SemaphoreType.DMA,
            pltpu.SemaphoreType.DMA,
        ],
    )
    out6 = run(column_embeddings, row_embeddings)
    return out6.transpose(0, 3, 5, 1, 2, 4).reshape(b, 2 * d, h, w)
